# Initial kernel scaffold; baseline (speedup 1.0000x reference)
import jax
import jax.numpy as jnp
from jax.experimental import pallas as pl


def kernel(x, W_feat, b_feat, tod_table, dow_table):
    raise NotImplementedError("write your pallas kernel here")


# speedup vs baseline: 1.9882x; 1.9882x over previous
import functools

import jax
import jax.numpy as jnp
from jax.experimental import pallas as pl
from jax.experimental.pallas import tpu as pltpu

STEPS_PER_DAY = 288
EMB = 64
BLK = 2048


def _body(x_ref, w_ref, b_ref, tod_ref, dow_ref, out_ref):
    xb = x_ref[...]
    x0 = xb[:, 0:1]
    feat = x0 * w_ref[...] + b_ref[...]
    tod_idx = (xb[:, 1:2] * STEPS_PER_DAY).astype(jnp.int32)
    dow_idx = xb[:, 2:3].astype(jnp.int32)
    iota = jax.lax.broadcasted_iota(jnp.int32, (BLK, STEPS_PER_DAY), 1)
    oh = (iota == tod_idx).astype(jnp.bfloat16)
    tod = jax.lax.dot_general(
        oh, tod_ref[...].astype(jnp.bfloat16), (((1,), (0,)), ((), ())),
        preferred_element_type=jnp.float32)
    dow = jnp.take_along_axis(
        dow_ref[...], jnp.broadcast_to(dow_idx, (BLK, EMB)), axis=0,
        mode="promise_in_bounds")
    out_ref[...] = jnp.concatenate([feat, tod, dow], axis=1)


@jax.jit
def kernel(x, W_feat, b_feat, tod_table, dow_table):
    B, T, C = x.shape
    n = B * T
    xf = x.reshape(n, C)
    grid = (n // BLK,)
    out = pl.pallas_call(
        _body,
        grid=grid,
        in_specs=[
            pl.BlockSpec((BLK, C), lambda i: (i, 0)),
            pl.BlockSpec((1, EMB), lambda i: (0, 0)),
            pl.BlockSpec((1, EMB), lambda i: (0, 0)),
            pl.BlockSpec((STEPS_PER_DAY, EMB), lambda i: (0, 0)),
            pl.BlockSpec((8, EMB), lambda i: (0, 0)),
        ],
        out_specs=pl.BlockSpec((BLK, 3 * EMB), lambda i: (i, 0)),
        out_shape=jax.ShapeDtypeStruct((n, 3 * EMB), jnp.float32),
        compiler_params=pltpu.CompilerParams(
            dimension_semantics=("arbitrary",),
        ),
    )(xf, W_feat, b_feat.reshape(1, EMB), tod_table,
      jnp.concatenate([dow_table, jnp.zeros((1, EMB), jnp.float32)], axis=0))
    return out.reshape(B, T, 3 * EMB)


# speedup vs baseline: 4.0808x; 2.0525x over previous
import functools

import jax
import jax.numpy as jnp
from jax import lax
from jax.experimental import pallas as pl
from jax.experimental.pallas import tpu as pltpu
from jax.experimental.pallas import tpu_sc as plsc

STEPS = 288
EMB = 64
NC, NS = 2, 16
NW = NC * NS
CHUNK = 256
NJ = EMB // 16
NBUF = 2


def _sc_body(x0_hbm, x1_hbm, x2_hbm, w_hbm, b_hbm, tod_hbm, dow_hbm, out_hbm,
             x0_v, x1_v, x2_v, idx1_v, idx2_v, stage_v, wb_v, tod_t, dow_t,
             in_sems, out_sems):
    n = x0_hbm.shape[0]
    npw = n // NW
    wid = lax.axis_index("s") * NC + lax.axis_index("c")
    base0 = wid * npw
    niter = npw // CHUNK

    pltpu.sync_copy(w_hbm, wb_v.at[0])
    pltpu.sync_copy(b_hbm, wb_v.at[1])
    pltpu.sync_copy(tod_hbm, tod_t)
    pltpu.sync_copy(dow_hbm, dow_t)
    w_regs = [wb_v[0, pl.ds(j * 16, 16)] for j in range(NJ)]
    b_regs = [wb_v[1, pl.ds(j * 16, 16)] for j in range(NJ)]
    lane = lax.iota(jnp.int32, 16)
    offs = [lane + (j * 16) for j in range(NJ)]

    def issue_in(g, b):
        base = base0 + g * CHUNK
        sl = pl.ds(base, CHUNK)
        pltpu.async_copy(x0_hbm.at[sl], x0_v.at[b], in_sems.at[b])
        pltpu.async_copy(x1_hbm.at[sl], x1_v.at[b], in_sems.at[b])
        pltpu.async_copy(x2_hbm.at[sl], x2_v.at[b], in_sems.at[b])

    def wait_in(b):
        pltpu.make_async_copy(x0_hbm.at[pl.ds(0, CHUNK)], x0_v.at[b],
                              in_sems.at[b]).wait()
        pltpu.make_async_copy(x1_hbm.at[pl.ds(0, CHUNK)], x1_v.at[b],
                              in_sems.at[b]).wait()
        pltpu.make_async_copy(x2_hbm.at[pl.ds(0, CHUNK)], x2_v.at[b],
                              in_sems.at[b]).wait()

    def wait_out(b):
        pltpu.make_async_copy(stage_v.at[b],
                              out_hbm.at[pl.ds(0, CHUNK)],
                              out_sems.at[b]).wait()

    def compute(g, b):
        @plsc.parallel_loop(0, CHUNK // 16, unroll=2)
        def idx_body(i):
            sl = pl.ds(i * 16, 16)
            v1 = x1_v[b, sl]
            idx1_v[b, sl] = (v1 * float(STEPS)).astype(jnp.int32) * EMB
            v2 = x2_v[b, sl]
            idx2_v[b, sl] = v2.astype(jnp.int32) * EMB

        @plsc.parallel_loop(0, CHUNK, unroll=4)
        def tok_body(t):
            t16 = jax.lax.broadcast_in_dim(t, (16,), ())
            s16 = plsc.load_gather(x0_v.at[b], [t16])
            tb16 = plsc.load_gather(idx1_v.at[b], [t16])
            db16 = plsc.load_gather(idx2_v.at[b], [t16])
            for j in range(NJ):
                stage_v[b, t, pl.ds(j * 16, 16)] = (
                    s16 * w_regs[j] + b_regs[j])
            for j in range(NJ):
                stage_v[b, t, pl.ds(EMB + j * 16, 16)] = plsc.load_gather(
                    tod_t, [tb16 + offs[j]])
            for j in range(NJ):
                stage_v[b, t, pl.ds(2 * EMB + j * 16, 16)] = plsc.load_gather(
                    dow_t, [db16 + offs[j]])

    def issue_out(g, b):
        base = base0 + g * CHUNK
        pltpu.async_copy(stage_v.at[b], out_hbm.at[pl.ds(base, CHUNK)],
                         out_sems.at[b])

    issue_in(0, 0)

    def pair_body(h, _):
        for b in range(NBUF):
            g = NBUF * h + b
            nb = (b + 1) % NBUF

            @pl.when(g + 1 < niter)
            def _():
                issue_in(g + 1, nb)
            wait_in(b)

            @pl.when(h > 0)
            def _():
                wait_out(b)
            compute(g, b)
            issue_out(g, b)
        return 0

    lax.fori_loop(0, niter // NBUF, pair_body, 0)
    for b in range(NBUF):
        wait_out(b)


@jax.jit
def kernel(x, W_feat, b_feat, tod_table, dow_table):
    B, T, C = x.shape
    n = B * T
    xf = x.reshape(n, C)
    x0 = xf[:, 0]
    x1 = xf[:, 1]
    x2 = xf[:, 2]
    mesh = plsc.VectorSubcoreMesh(core_axis_name="c", subcore_axis_name="s")
    f = pl.kernel(
        _sc_body,
        out_type=jax.ShapeDtypeStruct((n, 3 * EMB), jnp.float32),
        mesh=mesh,
        compiler_params=pltpu.CompilerParams(
            use_tc_tiling_on_sc=False, needs_layout_passes=False),
        scratch_types=[
            pltpu.VMEM((NBUF, CHUNK), jnp.float32),
            pltpu.VMEM((NBUF, CHUNK), jnp.float32),
            pltpu.VMEM((NBUF, CHUNK), jnp.float32),
            pltpu.VMEM((NBUF, CHUNK), jnp.int32),
            pltpu.VMEM((NBUF, CHUNK), jnp.int32),
            pltpu.VMEM((NBUF, CHUNK, 3 * EMB), jnp.float32),
            pltpu.VMEM((2, EMB), jnp.float32),
            pltpu.VMEM((STEPS * EMB,), jnp.float32),
            pltpu.VMEM((7 * EMB,), jnp.float32),
            pltpu.SemaphoreType.DMA((NBUF,)),
            pltpu.SemaphoreType.DMA((NBUF,)),
        ],
    )
    out = f(x0, x1, x2, W_feat.reshape(EMB), b_feat,
            tod_table.reshape(STEPS * EMB), dow_table.reshape(7 * EMB))
    return out.reshape(B, T, 3 * EMB)
